# baseline (speedup 1.0000x reference)
import functools

import jax
import jax.numpy as jnp
from jax.experimental import pallas as pl

B = 16
CHANNELS = 512
T = 2048
NUM_BITS = 16
T_BLK = 2048


B_BLK = 2


def _bitcodes_kernel(x_ref, cbcat_ref, sela_ref, out_ref, bits_ref):
    cbcat = cbcat_ref[...]
    sela = sela_ref[...]
    sela_hi = sela.astype(jnp.bfloat16)
    sela_lo = (sela - sela_hi.astype(jnp.float32)).astype(jnp.bfloat16)

    for r in range(B_BLK):
        xb = x_ref[r]
        s = jax.lax.dot_general(
            cbcat, xb, (((1,), (0,)), ((), ())),
            preferred_element_type=jnp.float32,
        )
        bits_t = s[NUM_BITS:, :] > s[:NUM_BITS, :]
        bits_tf = bits_t.astype(jnp.float32)
        eye = jnp.eye(NUM_BITS, dtype=jnp.bfloat16)
        bits_wide = jax.lax.dot_general(
            bits_tf.astype(jnp.bfloat16), eye, (((0,), (0,)), ((), ())),
            preferred_element_type=jnp.float32,
        )
        bits_ref[r] = bits_wide.astype(jnp.int32)

        bits_aug = jnp.concatenate(
            [bits_tf.astype(jnp.bfloat16),
             jnp.ones((1, T_BLK), jnp.bfloat16)], axis=0)
        dims = (((0,), (0,)), ((), ()))
        out_ref[r] = (
            jax.lax.dot_general(sela_hi, bits_aug, dims,
                                preferred_element_type=jnp.float32)
            + jax.lax.dot_general(sela_lo, bits_aug, dims,
                                  preferred_element_type=jnp.float32)
        )


@functools.partial(jax.jit, static_argnames=())
def kernel(x, codebook):
    cb0 = codebook[:, 0, :]
    cb1 = codebook[:, 1, :]
    cbcat = jnp.concatenate([cb0, cb1], axis=0)
    sela = jnp.concatenate([cb1 - cb0, jnp.sum(cb0, 0)[None]], axis=0)
    grid = (B // B_BLK, T // T_BLK)
    out, bits = pl.pallas_call(
        _bitcodes_kernel,
        grid=grid,
        in_specs=[
            pl.BlockSpec((B_BLK, CHANNELS, T_BLK), lambda b, t: (b, 0, t)),
            pl.BlockSpec((2 * NUM_BITS, CHANNELS), lambda b, t: (0, 0)),
            pl.BlockSpec((NUM_BITS + 1, CHANNELS), lambda b, t: (0, 0)),
        ],
        out_specs=[
            pl.BlockSpec((B_BLK, CHANNELS, T_BLK), lambda b, t: (b, 0, t)),
            pl.BlockSpec((B_BLK, T_BLK, NUM_BITS), lambda b, t: (b, t, 0)),
        ],
        out_shape=[
            jax.ShapeDtypeStruct((B, CHANNELS, T), jnp.float32),
            jax.ShapeDtypeStruct((B, T, NUM_BITS), jnp.int32),
        ],
    )(x, cbcat, sela)
    return out, bits


# speedup vs baseline: 1.0303x; 1.0303x over previous
import functools

import jax
import jax.numpy as jnp
from jax.experimental import pallas as pl

B = 16
CHANNELS = 512
T = 2048
NUM_BITS = 16
T_BLK = 2048


B_BLK = 2


def _bitcodes_kernel(x_ref, cbcat_ref, sela_ref, out_ref, bits_ref):
    cbcat = cbcat_ref[...]
    sela = sela_ref[...]

    for r in range(B_BLK):
        xb = x_ref[r]
        s = jax.lax.dot_general(
            cbcat, xb, (((1,), (0,)), ((), ())),
            preferred_element_type=jnp.float32,
        )
        bits_t = s[NUM_BITS:, :] > s[:NUM_BITS, :]
        bits_tf = bits_t.astype(jnp.float32)
        eye = jnp.eye(NUM_BITS, dtype=jnp.bfloat16)
        bits_wide = jax.lax.dot_general(
            bits_tf.astype(jnp.bfloat16), eye, (((0,), (0,)), ((), ())),
            preferred_element_type=jnp.float32,
        )
        bits_ref[r] = bits_wide.astype(jnp.int32)

        bits_aug = jnp.concatenate(
            [bits_tf,
             jnp.ones((1, T_BLK), jnp.float32)], axis=0)
        out_ref[r] = jax.lax.dot_general(
            sela, bits_aug, (((0,), (0,)), ((), ())),
            preferred_element_type=jnp.float32,
        )


@functools.partial(jax.jit, static_argnames=())
def kernel(x, codebook):
    cb0 = codebook[:, 0, :]
    cb1 = codebook[:, 1, :]
    cbcat = jnp.concatenate([cb0, cb1], axis=0)
    sela = jnp.concatenate([cb1 - cb0, jnp.sum(cb0, 0)[None]], axis=0)
    grid = (B // B_BLK, T // T_BLK)
    out, bits = pl.pallas_call(
        _bitcodes_kernel,
        grid=grid,
        in_specs=[
            pl.BlockSpec((B_BLK, CHANNELS, T_BLK), lambda b, t: (b, 0, t)),
            pl.BlockSpec((2 * NUM_BITS, CHANNELS), lambda b, t: (0, 0)),
            pl.BlockSpec((NUM_BITS + 1, CHANNELS), lambda b, t: (0, 0)),
        ],
        out_specs=[
            pl.BlockSpec((B_BLK, CHANNELS, T_BLK), lambda b, t: (b, 0, t)),
            pl.BlockSpec((B_BLK, T_BLK, NUM_BITS), lambda b, t: (b, t, 0)),
        ],
        out_shape=[
            jax.ShapeDtypeStruct((B, CHANNELS, T), jnp.float32),
            jax.ShapeDtypeStruct((B, T, NUM_BITS), jnp.int32),
        ],
    )(x, cbcat, sela)
    return out, bits
